# Initial kernel scaffold; baseline (speedup 1.0000x reference)
import jax
import jax.numpy as jnp
from jax.experimental import pallas as pl


def kernel(x, w1p, b1p, w2p, b2p):
    raise NotImplementedError("write your pallas kernel here")


# speedup vs baseline: 1.3828x; 1.3828x over previous
import jax
import jax.numpy as jnp
from jax.experimental import pallas as pl
from jax.experimental.pallas import tpu as pltpu

_OUT_DIM = 4


def _mlp_kernel(x_ref, w1_ref, b1_ref, w2_ref, b2_ref, o_ref):
    h = jnp.dot(x_ref[...], w1_ref[...], preferred_element_type=jnp.float32)
    h = jnp.maximum(h + b1_ref[...], 0.0)
    logits = jnp.dot(h, w2_ref[...], preferred_element_type=jnp.float32)
    o_ref[...] = (logits + b2_ref[...]).astype(o_ref.dtype)


def kernel(x, w1p, b1p, w2p, b2p):
    B, in_dim = x.shape
    hid_p = w1p.shape[1]

    w2s = w2p[:, :_OUT_DIM]
    b2s = b2p[:, :_OUT_DIM]

    tb = 4096
    bp = (B + tb - 1) // tb * tb
    if bp != B:
        x = jnp.pad(x, ((0, bp - B), (0, 0)))
    n_tiles = bp // tb

    out = pl.pallas_call(
        _mlp_kernel,
        out_shape=jax.ShapeDtypeStruct((bp, _OUT_DIM), jnp.float32),
        grid=(n_tiles,),
        in_specs=[
            pl.BlockSpec((tb, in_dim), lambda i: (i, 0)),
            pl.BlockSpec(w1p.shape, lambda i: (0, 0)),
            pl.BlockSpec(b1p.shape, lambda i: (0, 0)),
            pl.BlockSpec(w2s.shape, lambda i: (0, 0)),
            pl.BlockSpec(b2s.shape, lambda i: (0, 0)),
        ],
        out_specs=pl.BlockSpec((tb, _OUT_DIM), lambda i: (i, 0)),
        compiler_params=pltpu.CompilerParams(
            dimension_semantics=("parallel",)
        ),
    )(x, w1p, b1p, w2s, b2s)

    return out[:B]


# speedup vs baseline: 1.5271x; 1.1043x over previous
import jax
import jax.numpy as jnp
from jax.experimental import pallas as pl
from jax.experimental.pallas import tpu as pltpu

_OUT_DIM = 4


def _mlp_kernel(x_ref, w1_ref, b1_ref, w2_ref, b2_ref, o_ref):
    h = jnp.dot(x_ref[...], w1_ref[...], preferred_element_type=jnp.float32)
    h = jnp.maximum(h + b1_ref[...], 0.0)
    logits = jnp.dot(h, w2_ref[...], preferred_element_type=jnp.float32)
    o_ref[...] = (logits + b2_ref[...]).astype(o_ref.dtype)


def kernel(x, w1p, b1p, w2p, b2p):
    B, in_dim = x.shape
    hid_p = w1p.shape[1]

    w2s = w2p[:, :_OUT_DIM]
    b2s = b2p[:, :_OUT_DIM]

    tb = 16384
    bp = (B + tb - 1) // tb * tb
    if bp != B:
        x = jnp.pad(x, ((0, bp - B), (0, 0)))
    n_tiles = bp // tb

    out = pl.pallas_call(
        _mlp_kernel,
        out_shape=jax.ShapeDtypeStruct((bp, _OUT_DIM), jnp.float32),
        grid=(n_tiles,),
        in_specs=[
            pl.BlockSpec((tb, in_dim), lambda i: (i, 0)),
            pl.BlockSpec(w1p.shape, lambda i: (0, 0)),
            pl.BlockSpec(b1p.shape, lambda i: (0, 0)),
            pl.BlockSpec(w2s.shape, lambda i: (0, 0)),
            pl.BlockSpec(b2s.shape, lambda i: (0, 0)),
        ],
        out_specs=pl.BlockSpec((tb, _OUT_DIM), lambda i: (i, 0)),
        compiler_params=pltpu.CompilerParams(
            dimension_semantics=("parallel",)
        ),
    )(x, w1p, b1p, w2s, b2s)

    return out[:B]
